# baseline (speedup 1.0000x reference)
import functools

import jax
import jax.numpy as jnp
from jax import lax
from jax.experimental import pallas as pl
from jax.experimental.pallas import tpu as pltpu
from jax.experimental.pallas import tpu_sc as plsc

V = 1000
VP = 1008
D = 64
NC = 2
NS = 16
NW = NC * NS
CHUNK = 64


def _table_body(e_ref, wt_ref, b_ref, out_ref):
    out_ref[...] = (
        jnp.dot(e_ref[...], wt_ref[...],
                preferred_element_type=jnp.float32,
                precision=lax.Precision.HIGHEST)
        + b_ref[...]
    )


def _make_table(embed_weight, wt, bias2d):
    return pl.pallas_call(
        _table_body,
        out_shape=jax.ShapeDtypeStruct((V, VP), jnp.float32),
    )(embed_weight, wt, bias2d)


def _gather_body(nchunk, table_hbm, idx_hbm, out_hbm, idx_v, rows_v, sem):
    wid = lax.axis_index("s") * NC + lax.axis_index("c")

    def chunk_step(j, carry):
        base = (wid * nchunk + j) * CHUNK
        pltpu.sync_copy(idx_hbm.at[pl.ds(base, CHUNK)], idx_v)
        pltpu.async_copy(table_hbm.at[idx_v], rows_v, sem).wait()
        pltpu.sync_copy(rows_v.at[:, pl.ds(0, V)],
                        out_hbm.at[pl.ds(base, CHUNK)])
        return carry

    lax.fori_loop(0, nchunk, chunk_step, 0)


def _gather_rows(table, idx, n_rows, nchunk):
    mesh = plsc.VectorSubcoreMesh(
        core_axis_name="c", subcore_axis_name="s",
        num_cores=NC, num_subcores=NS)
    run = pl.kernel(
        functools.partial(_gather_body, nchunk),
        out_type=jax.ShapeDtypeStruct((n_rows, V), jnp.float32),
        mesh=mesh,
        compiler_params=pltpu.CompilerParams(use_tc_tiling_on_sc=False),
        scratch_types=[
            pltpu.VMEM((CHUNK,), jnp.int32),
            pltpu.VMEM((CHUNK, VP), jnp.float32),
            pltpu.SemaphoreType.DMA,
        ],
    )
    return run(table, idx)


def kernel(token_ids, embed_weight, proj_weight, proj_bias):
    B, T = token_ids.shape
    n_rows = B * T
    assert n_rows % (NW * CHUNK) == 0
    nchunk = n_rows // (NW * CHUNK)

    wt = jnp.pad(proj_weight.T, ((0, 0), (0, VP - V)))
    bias2d = jnp.pad(proj_bias.reshape(1, V), ((0, 0), (0, VP - V)))
    table = _make_table(embed_weight, wt, bias2d)

    idx = token_ids.reshape(n_rows).astype(jnp.int32)
    flat = _gather_rows(table, idx, n_rows, nchunk)
    return flat.reshape(B, T, V)


# speedup vs baseline: 1.0240x; 1.0240x over previous
import functools

import jax
import jax.numpy as jnp
from jax import lax
from jax.experimental import pallas as pl
from jax.experimental.pallas import tpu as pltpu
from jax.experimental.pallas import tpu_sc as plsc

V = 1000
VP = 1008
D = 64
NC = 2
NS = 16
NW = NC * NS
CHUNK = 40


def _table_body(e_ref, wt_ref, b_ref, out_ref):
    out_ref[...] = (
        jnp.dot(e_ref[...], wt_ref[...],
                preferred_element_type=jnp.float32,
                precision=lax.Precision.HIGHEST)
        + b_ref[...]
    )


def _make_table(embed_weight, wt, bias2d):
    return pl.pallas_call(
        _table_body,
        out_shape=jax.ShapeDtypeStruct((V, VP), jnp.float32),
    )(embed_weight, wt, bias2d)


def _gather_body(nchunk, table_hbm, idx_hbm, out_hbm,
                 idx_v, rows0, rows1, semg0, semg1):
    wid = lax.axis_index("s") * NC + lax.axis_index("c")
    bpw = nchunk * CHUNK
    base = wid * bpw

    pltpu.sync_copy(idx_hbm.at[pl.ds(base, bpw)], idx_v)

    def fire(j, rows, sem):
        src = table_hbm.at[idx_v.at[pl.ds(j * CHUNK, CHUNK)]]
        pltpu.async_copy(src, rows, sem)

    def drain_and_write(j, rows, sem):
        pltpu.make_async_copy(table_hbm.at[idx_v.at[pl.ds(0, CHUNK)]],
                              rows, sem).wait()
        pltpu.sync_copy(rows.at[:, pl.ds(0, V)],
                        out_hbm.at[pl.ds(base + j * CHUNK, CHUNK)])

    fire(0, rows0, semg0)
    fire(1, rows1, semg1)

    def pair_step(p, carry):
        j = 2 * p
        drain_and_write(j, rows0, semg0)
        fire(j + 2, rows0, semg0)
        drain_and_write(j + 1, rows1, semg1)
        fire(j + 3, rows1, semg1)
        return carry

    lax.fori_loop(0, nchunk // 2 - 1, pair_step, 0)
    drain_and_write(nchunk - 2, rows0, semg0)
    drain_and_write(nchunk - 1, rows1, semg1)


def _gather_rows(table, idx, n_rows, nchunk):
    mesh = plsc.VectorSubcoreMesh(
        core_axis_name="c", subcore_axis_name="s",
        num_cores=NC, num_subcores=NS)
    run = pl.kernel(
        functools.partial(_gather_body, nchunk),
        out_type=jax.ShapeDtypeStruct((n_rows, V), jnp.float32),
        mesh=mesh,
        compiler_params=pltpu.CompilerParams(use_tc_tiling_on_sc=False),
        scratch_types=[
            pltpu.VMEM((nchunk * CHUNK,), jnp.int32),
            pltpu.VMEM((CHUNK, VP), jnp.float32),
            pltpu.VMEM((CHUNK, VP), jnp.float32),
            pltpu.SemaphoreType.DMA,
            pltpu.SemaphoreType.DMA,
        ],
    )
    return run(table, idx)


def kernel(token_ids, embed_weight, proj_weight, proj_bias):
    B, T = token_ids.shape
    n_rows = B * T
    assert n_rows % (NW * CHUNK) == 0
    nchunk = n_rows // (NW * CHUNK)

    wt = jnp.pad(proj_weight.T, ((0, 0), (0, VP - V)))
    bias2d = jnp.pad(proj_bias.reshape(1, V), ((0, 0), (0, VP - V)))
    table = _make_table(embed_weight, wt, bias2d)

    idx = token_ids.reshape(n_rows).astype(jnp.int32)
    flat = _gather_rows(table, idx, n_rows, nchunk)
    return flat.reshape(B, T, V)
